# Initial kernel scaffold; baseline (speedup 1.0000x reference)
import jax
import jax.numpy as jnp
from jax.experimental import pallas as pl


def kernel(inputs, emb_table):
    raise NotImplementedError("write your pallas kernel here")


# speedup vs baseline: 1.4884x; 1.4884x over previous
import jax
import jax.numpy as jnp
from jax.experimental import pallas as pl

S_BLK = 512


def _add_body(x_ref, t_ref, o_ref):
    o_ref[...] = x_ref[...] + t_ref[...][None]


def kernel(inputs, emb_table):
    B, S, D = inputs.shape
    n_s = S // S_BLK
    return pl.pallas_call(
        _add_body,
        grid=(n_s, B),
        in_specs=[
            pl.BlockSpec((1, S_BLK, D), lambda s, b: (b, s, 0)),
            pl.BlockSpec((S_BLK, D), lambda s, b: (s, 0)),
        ],
        out_specs=pl.BlockSpec((1, S_BLK, D), lambda s, b: (b, s, 0)),
        out_shape=jax.ShapeDtypeStruct((B, S, D), inputs.dtype),
    )(inputs, emb_table)
